# speedup vs baseline: 1.5239x; 1.5239x over previous
import jax
import jax.numpy as jnp
from jax.experimental import pallas as pl
from jax.experimental.pallas import tpu as pltpu

B = 16384
OBS = 128
GENC = 64
HID = 128
SKILL = 64
NF = 8
INP = GENC + OBS + OBS

BM = 1024


def _fused_kernel(g_ref, s_ref, n_ref, W1_ref, b1_ref, W2_ref, b2_ref,
                  W3_ref, b3_ref, out_ref):
    g = g_ref[...]
    s = s_ref[...]
    n = n_ref[...]
    acc = jnp.zeros((g.shape[0], SKILL), jnp.float32)
    for i in range(NF):
        h = (g @ W1_ref[i, :GENC, :]
             + s @ W1_ref[i, GENC:GENC + OBS, :]
             + n @ W1_ref[i, GENC + OBS:, :]
             + b1_ref[i])
        h = jnp.maximum(h, 0.0)
        h = jnp.maximum(h @ W2_ref[i] + b2_ref[i], 0.0)
        o = h @ W3_ref[i] + b3_ref[i]
        mask = g[:, i:i + 1] == 1.0
        acc = jnp.where(mask, o, acc)
    out_ref[...] = acc


def kernel(graph, state, next_state, W1, b1, W2, b2, W3, b3):
    grid = (B // BM,)
    row_spec = lambda w: pl.BlockSpec((BM, w), lambda i: (i, 0))
    full = lambda shape: pl.BlockSpec(shape, lambda i: tuple(0 for _ in shape))
    return pl.pallas_call(
        _fused_kernel,
        grid=grid,
        in_specs=[
            row_spec(GENC),
            row_spec(OBS),
            row_spec(OBS),
            full((NF, INP, HID)),
            full((NF, HID)),
            full((NF, HID, HID)),
            full((NF, HID)),
            full((NF, HID, SKILL)),
            full((NF, SKILL)),
        ],
        out_specs=row_spec(SKILL),
        out_shape=jax.ShapeDtypeStruct((B, SKILL), jnp.float32),
    )(graph, state, next_state, W1, b1, W2, b2, W3, b3)
